# baseline (speedup 1.0000x reference)
import functools

import jax
import jax.numpy as jnp
import numpy as np
from jax import lax
from jax.experimental import pallas as pl
from jax.experimental.pallas import tpu as pltpu
from jax.experimental.pallas import tpu_sc as plsc

N = 10000
NPAD = 10240
NC = 40
FD = 256
FH = 256
DH = 64
HID = 512
E = 160000
MAXD = 256
NE = 8
CFGS = [("logits", "features"), ("features",), ("logits",),
        ("logits", "features", "degrees"), ("features", "degrees"),
        ("degrees",), ("logits", "degrees"), ("logits", "features")]
DEG_EXPERTS = [3, 4, 5, 6]
FEAT_EXPERTS = [0, 1, 3, 4, 7]
C1 = 640
C2 = 384

NBLK = 1024
WBLK = 128
NB1 = C1 // WBLK
NB2 = C2 // WBLK
EPAD = 163840
NCH = EPAD // 16 // 128
TROWS = NPAD // 16


def _stage_e_body(u0, u1, u2, u3, u4s, rin_ref, rout_ref, w1_ref, b1_ref,
                  w2_ref, z_ref):
    bf16 = jnp.bfloat16
    rin = rin_ref[...]
    us = [(u0[...] * rin).astype(bf16), (u1[...] * rin).astype(bf16),
          (u2[...] * rin).astype(bf16), (u3[...] * rin).astype(bf16),
          ((u4s[0] + u4s[1]) * rin).astype(bf16)]
    outs = []
    for e in range(NE):
        h = jnp.dot(us[0], w1_ref[e, 0:WBLK].astype(bf16),
                    preferred_element_type=jnp.float32)
        for b in range(1, NB1):
            h = h + jnp.dot(us[b], w1_ref[e, b * WBLK:(b + 1) * WBLK].astype(bf16),
                            preferred_element_type=jnp.float32)
        h = jnp.maximum(h + b1_ref[0, e][None, :], 0.0)
        z = jnp.dot(h.astype(bf16), w2_ref[e].astype(bf16),
                    preferred_element_type=jnp.float32)
        outs.append(z)
    outs.append(jnp.zeros((NBLK, C2 - NE * NC), jnp.float32))
    z_ref[...] = jnp.concatenate(outs, axis=1) * rout_ref[...]


def _stage_e(ublocks, usplit, rin, rout, w1eff, b1s, w2s):
    grid = NPAD // NBLK
    ub_spec = pl.BlockSpec((NBLK, WBLK), lambda i: (i, 0))
    return pl.pallas_call(
        _stage_e_body,
        grid=(grid,),
        in_specs=[ub_spec] * (NB1 - 1) + [
            pl.BlockSpec((2, NBLK, WBLK), lambda i: (0, i, 0)),
            pl.BlockSpec((NBLK, 1), lambda i: (i, 0)),
            pl.BlockSpec((NBLK, 1), lambda i: (i, 0)),
            pl.BlockSpec((NE, C1, HID), lambda i: (0, 0, 0)),
            pl.BlockSpec((1, NE, HID), lambda i: (0, 0, 0)),
            pl.BlockSpec((NE, HID, NC), lambda i: (0, 0, 0)),
        ],
        out_specs=pl.BlockSpec((NBLK, C2), lambda i: (i, 0)),
        out_shape=jax.ShapeDtypeStruct((NPAD, C2), jnp.float32),
    )(*ublocks, usplit, rin, rout, w1eff, b1s, w2s)


def _make_agg(nfull):
    mesh = plsc.VectorSubcoreMesh(core_axis_name="c", subcore_axis_name="s")
    f32 = jnp.float32
    w = WBLK
    nblocks = nfull + 1

    @functools.partial(
        pl.kernel,
        mesh=mesh,
        out_type=[jax.ShapeDtypeStruct((NPAD, w), f32)] * nfull
        + [jax.ShapeDtypeStruct((2, NPAD, w), f32)],
        scratch_types=[
            pltpu.VMEM((NCH // 2, 128), jnp.int32),
            pltpu.VMEM((NCH // 2, 128), jnp.int32),
            pltpu.VMEM((128, w), f32),
            pltpu.VMEM((128, w), f32),
            pltpu.VMEM_SHARED((NPAD, w), f32),
            pltpu.SemaphoreType.DMA,
        ],
    )
    def agg(*refs):
        xs = refs[:nblocks]
        srcp = refs[nblocks]
        dstp = refs[nblocks + 1]
        outs = refs[nblocks + 2:nblocks + 2 + nblocks]
        src_v, dst_v, buf_a, buf_b, acc, gsem = refs[nblocks + 2 + nblocks:]
        cid = lax.axis_index("c")
        sid = lax.axis_index("s")
        nh = NCH // 2

        def process(x_hbm, out_rows, h_pred):
            @pl.loop(0, 128)
            def _zrow(j):
                for k in range(w // 16):
                    buf_a[j, pl.ds(k * 16, 16)] = jnp.zeros((16,), f32)

            for r in range(TROWS // 128):
                pltpu.sync_copy(buf_a,
                                acc.at[pl.ds(sid * TROWS + r * 128, 128)])
            plsc.subcore_barrier()

            for h in range(2):
                @pl.when(h_pred[h])
                def _half():
                    pltpu.sync_copy(
                        srcp.at[sid, pl.ds(h * nh, nh)], src_v)
                    pltpu.sync_copy(
                        dstp.at[sid, pl.ds(h * nh, nh)], dst_v)

                    pltpu.async_copy(x_hbm.at[src_v.at[0]], buf_a, gsem)

                    @pl.loop(0, nh, step=2)
                    def _chunk(j):
                        pltpu.make_async_copy(x_hbm.at[src_v.at[j]], buf_a,
                                              gsem).wait()
                        pltpu.async_copy(x_hbm.at[src_v.at[j + 1]], buf_b,
                                         gsem)
                        pltpu.sync_copy(buf_a, acc.at[dst_v.at[j]], add=True)
                        pltpu.make_async_copy(x_hbm.at[src_v.at[j + 1]],
                                              buf_b, gsem).wait()

                        @pl.when(j + 2 < nh)
                        def _g2():
                            pltpu.async_copy(x_hbm.at[src_v.at[j + 2]],
                                             buf_a, gsem)

                        pltpu.sync_copy(buf_b, acc.at[dst_v.at[j + 1]],
                                        add=True)

            plsc.subcore_barrier()
            pltpu.sync_copy(acc.at[pl.ds(sid * TROWS, TROWS)],
                            out_rows.at[pl.ds(sid * TROWS, TROWS)])
            plsc.subcore_barrier()

        for b in range(nfull):
            @pl.when(cid == b % 2)
            def _full():
                process(xs[b], outs[b], [True, True])

        process(xs[nfull], outs[nfull].at[cid], [cid == 0, cid == 1])

    return agg


_AGG1 = _make_agg(NB1 - 1)
_AGG2 = _make_agg(NB2 - 1)


def kernel(logits, features, edge_index, gate_Wp, gate_bp, w_gate, expert_params):
    f32 = jnp.float32
    src = edge_index[0]
    dst = edge_index[1]

    ones_e = jnp.ones((E,), f32)
    deg_in = jnp.zeros((N,), f32).at[dst].add(ones_e)
    deg_out = jnp.zeros((N,), f32).at[src].add(ones_e)
    deg_in_f = jnp.maximum(deg_in, 1.0)
    deg_out_f = jnp.maximum(deg_out, 1.0)
    rin = jax.lax.rsqrt(deg_in_f)
    rout = jax.lax.rsqrt(deg_out_f)
    degrees = jnp.clip(deg_in + deg_out, 0, MAXD - 1).astype(jnp.int32)

    gate_h = features @ gate_Wp + gate_bp
    gate_logits = logits @ w_gate[:NC] + gate_h @ w_gate[NC:]
    top_vals, top_idx = jax.lax.top_k(gate_logits, 2)
    top_gates = jax.nn.softmax(top_vals, axis=-1)
    gates = jnp.zeros((N, NE), f32).at[jnp.arange(N)[:, None], top_idx].set(top_gates)
    importance = gates.sum(0)
    load = (gates > 0).sum(0).astype(f32)

    def cv2(x):
        return x.var() / (x.mean() ** 2 + 1e-10)

    aux_loss = 0.01 * (cv2(importance) + cv2(load))

    cols = [logits, features]
    for e in DEG_EXPERTS:
        cols.append(expert_params[e]["deg_table"][degrees])
    cols.append(jnp.ones((N, 1), f32))
    cols.append(jnp.zeros((N, C1 - 553), f32))
    x = jnp.concatenate(cols, axis=1) * rout[:, None]
    x = jnp.pad(x, ((0, NPAD - N), (0, 0)))

    srcp = jnp.pad(src, (0, EPAD - E), constant_values=N).reshape(16, NCH, 128)
    dstp = jnp.pad(dst, (0, EPAD - E), constant_values=N).reshape(16, NCH, 128)
    xblocks = [x[:, b * WBLK:(b + 1) * WBLK] for b in range(NB1)]
    *ublocks, usplit = _AGG1(*xblocks, srcp, dstp)

    w1eff = []
    b1s = []
    w2s = []
    for e, (cfg, p) in enumerate(zip(CFGS, expert_params)):
        w1 = p["W1"]
        off = 0
        rows = jnp.zeros((C1, HID), f32)
        if "logits" in cfg:
            rows = rows.at[0:NC].set(w1[off:off + NC])
            off += NC
        if "features" in cfg:
            w1f = w1[off:off + FH]
            off += FH
            rows = rows.at[40:40 + FD].set(p["Wf"] @ w1f)
            rows = rows.at[552].set(p["bf"] @ w1f)
        if "degrees" in cfg:
            slot = DEG_EXPERTS.index(e)
            rows = rows.at[296 + 64 * slot:296 + 64 * slot + DH].set(w1[off:off + DH])
            off += DH
        w1eff.append(rows)
        b1s.append(p["b1"])
        w2s.append(p["W2"])
    w1eff = jnp.stack(w1eff)
    b1s = jnp.stack(b1s)[None]
    w2s = jnp.stack(w2s)

    rin_pad = jnp.pad(rin, (0, NPAD - N))[:, None]
    rout_pad = jnp.pad(rout, (0, NPAD - N))[:, None]

    z = _stage_e(ublocks, usplit, rin_pad, rout_pad, w1eff, b1s, w2s)

    zblocks = [z[:, b * WBLK:(b + 1) * WBLK] for b in range(NB2)]
    *yblocks, ysplit = _AGG2(*zblocks, srcp, dstp)
    y_raw = jnp.concatenate(yblocks + [ysplit[0] + ysplit[1]], axis=1)
    yg = (y_raw * rin_pad)[:N, :NE * NC]

    b2s = jnp.stack([p["b2"] for p in expert_params])
    yg = yg.reshape(N, NE, NC)
    y = jnp.einsum("ne,nec->nc", gates, yg) + gates @ b2s
    return y, aux_loss


# speedup vs baseline: 1.0742x; 1.0742x over previous
import functools

import jax
import jax.numpy as jnp
import numpy as np
from jax import lax
from jax.experimental import pallas as pl
from jax.experimental.pallas import tpu as pltpu
from jax.experimental.pallas import tpu_sc as plsc

N = 10000
NPAD = 10240
NC = 40
FD = 256
FH = 256
DH = 64
HID = 512
E = 160000
MAXD = 256
NE = 8
CFGS = [("logits", "features"), ("features",), ("logits",),
        ("logits", "features", "degrees"), ("features", "degrees"),
        ("degrees",), ("logits", "degrees"), ("logits", "features")]
DEG_EXPERTS = [3, 4, 5, 6]
FEAT_EXPERTS = [0, 1, 3, 4, 7]
C1 = 640
C2 = 384

NBLK = 1024
WBLK = 128
NB1 = C1 // WBLK
NB2 = C2 // WBLK
EPAD = 163840
NCH = EPAD // 16 // 128
TROWS = NPAD // 16


def _stage_e_body(u0, u1, u2, u3, u4s, rin_ref, rout_ref, w1_ref, b1_ref,
                  w2_ref, z_ref):
    bf16 = jnp.bfloat16
    rin = rin_ref[...]
    us = [(u0[...] * rin).astype(bf16), (u1[...] * rin).astype(bf16),
          (u2[...] * rin).astype(bf16), (u3[...] * rin).astype(bf16),
          ((u4s[0] + u4s[1]) * rin).astype(bf16)]
    outs = []
    for e in range(NE):
        h = jnp.dot(us[0], w1_ref[e, 0:WBLK].astype(bf16),
                    preferred_element_type=jnp.float32)
        for b in range(1, NB1):
            h = h + jnp.dot(us[b], w1_ref[e, b * WBLK:(b + 1) * WBLK].astype(bf16),
                            preferred_element_type=jnp.float32)
        h = jnp.maximum(h + b1_ref[0, e][None, :], 0.0)
        z = jnp.dot(h.astype(bf16), w2_ref[e].astype(bf16),
                    preferred_element_type=jnp.float32)
        outs.append(z)
    outs.append(jnp.zeros((NBLK, C2 - NE * NC), jnp.float32))
    z_ref[...] = jnp.concatenate(outs, axis=1) * rout_ref[...]


def _stage_e(ublocks, usplit, rin, rout, w1eff, b1s, w2s):
    grid = NPAD // NBLK
    ub_spec = pl.BlockSpec((NBLK, WBLK), lambda i: (i, 0))
    return pl.pallas_call(
        _stage_e_body,
        grid=(grid,),
        in_specs=[ub_spec] * (NB1 - 1) + [
            pl.BlockSpec((2, NBLK, WBLK), lambda i: (0, i, 0)),
            pl.BlockSpec((NBLK, 1), lambda i: (i, 0)),
            pl.BlockSpec((NBLK, 1), lambda i: (i, 0)),
            pl.BlockSpec((NE, C1, HID), lambda i: (0, 0, 0)),
            pl.BlockSpec((1, NE, HID), lambda i: (0, 0, 0)),
            pl.BlockSpec((NE, HID, NC), lambda i: (0, 0, 0)),
        ],
        out_specs=pl.BlockSpec((NBLK, C2), lambda i: (i, 0)),
        out_shape=jax.ShapeDtypeStruct((NPAD, C2), jnp.float32),
    )(*ublocks, usplit, rin, rout, w1eff, b1s, w2s)


def _make_agg(nfull):
    mesh = plsc.VectorSubcoreMesh(core_axis_name="c", subcore_axis_name="s")
    f32 = jnp.float32
    w = WBLK
    nblocks = nfull + 1

    @functools.partial(
        pl.kernel,
        mesh=mesh,
        out_type=[jax.ShapeDtypeStruct((NPAD, w), f32)] * nfull
        + [jax.ShapeDtypeStruct((2, NPAD, w), f32)],
        scratch_types=[
            pltpu.VMEM((NCH // 2, 128), jnp.int32),
            pltpu.VMEM((NCH // 2, 128), jnp.int32),
            pltpu.VMEM((128, w), f32),
            pltpu.VMEM((128, w), f32),
            pltpu.VMEM_SHARED((NPAD, w), f32),
            pltpu.SemaphoreType.DMA,
        ],
    )
    def agg(*refs):
        xs = refs[:nblocks]
        srcp = refs[nblocks]
        dstp = refs[nblocks + 1]
        outs = refs[nblocks + 2:nblocks + 2 + nblocks]
        src_v, dst_v, buf_a, buf_b, acc, gsem = refs[nblocks + 2 + nblocks:]
        cid = lax.axis_index("c")
        sid = lax.axis_index("s")
        nh = NCH // 2

        def process(x_hbm, out_rows, h_pred):
            @pl.loop(0, 128)
            def _zrow(j):
                for k in range(w // 16):
                    buf_a[j, pl.ds(k * 16, 16)] = jnp.zeros((16,), f32)

            for r in range(TROWS // 128):
                pltpu.sync_copy(buf_a,
                                acc.at[pl.ds(sid * TROWS + r * 128, 128)])
            plsc.subcore_barrier()

            for h in range(2):
                @pl.when(h_pred[h])
                def _half():
                    pltpu.sync_copy(
                        srcp.at[sid, pl.ds(h * nh, nh)], src_v)
                    pltpu.sync_copy(
                        dstp.at[sid, pl.ds(h * nh, nh)], dst_v)

                    pltpu.async_copy(x_hbm.at[src_v.at[0]], buf_a, gsem)

                    @pl.loop(0, nh, step=2)
                    def _chunk(j):
                        pltpu.make_async_copy(x_hbm.at[src_v.at[j]], buf_a,
                                              gsem).wait()
                        pltpu.async_copy(x_hbm.at[src_v.at[j + 1]], buf_b,
                                         gsem)
                        pltpu.sync_copy(buf_a, acc.at[dst_v.at[j]], add=True)
                        pltpu.make_async_copy(x_hbm.at[src_v.at[j + 1]],
                                              buf_b, gsem).wait()

                        @pl.when(j + 2 < nh)
                        def _g2():
                            pltpu.async_copy(x_hbm.at[src_v.at[j + 2]],
                                             buf_a, gsem)

                        pltpu.sync_copy(buf_b, acc.at[dst_v.at[j + 1]],
                                        add=True)

            plsc.subcore_barrier()
            pltpu.sync_copy(acc.at[pl.ds(sid * TROWS, TROWS)],
                            out_rows.at[pl.ds(sid * TROWS, TROWS)])
            plsc.subcore_barrier()

        for b in range(nfull):
            @pl.when(cid == b % 2)
            def _full():
                process(xs[b], outs[b], [True, True])

        process(xs[nfull], outs[nfull].at[cid], [cid == 0, cid == 1])

    return agg


_AGG_CACHE = {}


def _agg(nfull):
    if nfull not in _AGG_CACHE:
        _AGG_CACHE[nfull] = _make_agg(nfull)
    return _AGG_CACHE[nfull]


def _build_body(pin, pout, lg, gl_ref, ft, dtc, xb0, xb1, xb2, xb3,
                xb4, rin_o, rout_o, gates_o, il_o):
    f32 = jnp.float32
    i = pl.program_id(0)
    din = jnp.sum(pin[...], axis=0)
    dout = jnp.sum(pout[...], axis=0)
    rin = lax.rsqrt(jnp.maximum(din, 1.0))
    rout = lax.rsqrt(jnp.maximum(dout, 1.0))
    rin_o[...] = rin[:, None]
    rout_o[...] = rout[:, None]
    deg = jnp.clip(din + dout, 0.0, float(MAXD - 1)).astype(jnp.int32)
    oh = (deg[:, None] == lax.broadcasted_iota(jnp.int32, (NBLK, MAXD), 1)
          ).astype(f32)
    demb = jnp.dot(oh, dtc[...], preferred_element_type=f32,
                   precision=lax.Precision.HIGHEST)

    gl_ = gl_ref[...]
    iota8 = lax.broadcasted_iota(jnp.int32, (NBLK, NE), 1)
    m1 = jnp.max(gl_, axis=1, keepdims=True)
    i1 = jnp.min(jnp.where(gl_ == m1, iota8, NE), axis=1, keepdims=True)
    neg = jnp.float32(-jnp.inf)
    masked = jnp.where(iota8 == i1, neg, gl_)
    m2 = jnp.max(masked, axis=1, keepdims=True)
    i2 = jnp.min(jnp.where(masked == m2, iota8, NE), axis=1, keepdims=True)
    ex = jnp.exp(m2 - m1)
    g1 = 1.0 / (1.0 + ex)
    g2 = ex / (1.0 + ex)
    gates = jnp.where(iota8 == i1, g1, 0.0) + jnp.where(iota8 == i2, g2, 0.0)
    rowid = i * NBLK + lax.broadcasted_iota(jnp.int32, (NBLK, 1), 0)
    gates = jnp.where(rowid < N, gates, 0.0)
    gates_o[...] = gates
    pimp = jnp.sum(gates, axis=0)[None]
    pload = jnp.sum((gates > 0.0).astype(f32), axis=0)[None]
    part = jnp.concatenate([pimp, pload], axis=0)

    @pl.when(i == 0)
    def _init():
        il_o[...] = part

    @pl.when(i > 0)
    def _acc():
        il_o[...] += part

    ro = rout[:, None]
    xb0[...] = jnp.concatenate(
        [lg[...], jnp.ones((NBLK, 1), f32),
         jnp.zeros((NBLK, WBLK - NC - 1), f32)], axis=1) * ro
    xb1[...] = ft[:, 0:128] * ro
    xb2[...] = ft[:, 128:256] * ro
    xb3[...] = demb[:, 0:128] * ro
    xb4[...] = demb[:, 128:256] * ro


def _build(pin, pout, lg, gl, ft, dtc):
    f32 = jnp.float32
    grid = NPAD // NBLK
    row = lambda i: (i, 0)
    full = lambda i: (0, 0)
    return pl.pallas_call(
        _build_body,
        grid=(grid,),
        in_specs=[
            pl.BlockSpec((1, NBLK), lambda i: (0, i)),
            pl.BlockSpec((1, NBLK), lambda i: (0, i)),
            pl.BlockSpec((NBLK, NC), row),
            pl.BlockSpec((NBLK, NE), row),
            pl.BlockSpec((NBLK, FD), row),
            pl.BlockSpec((MAXD, 4 * DH), full),
        ],
        out_specs=[pl.BlockSpec((NBLK, WBLK), row)] * 5 + [
            pl.BlockSpec((NBLK, 1), row),
            pl.BlockSpec((NBLK, 1), row),
            pl.BlockSpec((NBLK, NE), row),
            pl.BlockSpec((2, NE), full),
        ],
        out_shape=[jax.ShapeDtypeStruct((NPAD, WBLK), f32)] * 5 + [
            jax.ShapeDtypeStruct((NPAD, 1), f32),
            jax.ShapeDtypeStruct((NPAD, 1), f32),
            jax.ShapeDtypeStruct((NPAD, NE), f32),
            jax.ShapeDtypeStruct((2, NE), f32),
        ],
    )(pin, pout, lg, gl, ft, dtc)


NBLK2 = 1000


def _combine_body(yb0, yb1, ys, gates, rin, b2s, il, y_o, aux_o):
    f32 = jnp.float32
    i = pl.program_id(0)
    ycat = jnp.concatenate([yb0[...], yb1[...], ys[0] + ys[1]],
                           axis=1) * rin[...]
    g = gates[...]
    acc = jnp.dot(g, b2s[...], preferred_element_type=f32)
    for e in range(NE):
        acc = acc + g[:, e:e + 1] * ycat[:, e * NC:(e + 1) * NC]
    y_o[...] = acc

    @pl.when(i == 0)
    def _aux():
        def cv2(x):
            mu = jnp.mean(x)
            var = jnp.mean(x * x) - mu * mu
            return var / (mu * mu + 1e-10)

        aux_o[...] = (0.01 * (cv2(il[0]) + cv2(il[1])))[None, None]


def _combine(yb0, yb1, ys, gates, rin, b2s, il):
    f32 = jnp.float32
    grid = N // NBLK2
    row = lambda i: (i, 0)
    full = lambda i: (0, 0)
    return pl.pallas_call(
        _combine_body,
        grid=(grid,),
        in_specs=[
            pl.BlockSpec((NBLK2, WBLK), row),
            pl.BlockSpec((NBLK2, WBLK), row),
            pl.BlockSpec((2, NBLK2, WBLK), lambda i: (0, i, 0)),
            pl.BlockSpec((NBLK2, NE), row),
            pl.BlockSpec((NBLK2, 1), row),
            pl.BlockSpec((NE, NC), full),
            pl.BlockSpec((2, NE), full),
        ],
        out_specs=[pl.BlockSpec((NBLK2, NC), row),
                   pl.BlockSpec((1, 1), full)],
        out_shape=[jax.ShapeDtypeStruct((N, NC), f32),
                   jax.ShapeDtypeStruct((1, 1), f32)],
    )(yb0, yb1, ys, gates, rin, b2s, il)


def kernel(logits, features, edge_index, gate_Wp, gate_bp, w_gate, expert_params):
    f32 = jnp.float32
    src = edge_index[0]
    dst = edge_index[1]

    srcp = jnp.pad(src, (0, EPAD - E), constant_values=N).reshape(16, NCH, 128)
    dstp = jnp.pad(dst, (0, EPAD - E), constant_values=N).reshape(16, NCH, 128)
    ones_e = jnp.ones((E,), f32)
    pin = jnp.pad(jnp.zeros((N,), f32).at[dst].add(ones_e),
                  (0, NPAD - N))[None]
    pout = jnp.pad(jnp.zeros((N,), f32).at[src].add(ones_e),
                   (0, NPAD - N))[None]

    lg = jnp.pad(logits, ((0, NPAD - N), (0, 0)))
    ft = jnp.pad(features, ((0, NPAD - N), (0, 0)))
    dtc = jnp.concatenate([expert_params[e]["deg_table"]
                           for e in DEG_EXPERTS], axis=1)
    gate_h = features @ gate_Wp + gate_bp
    gl = jnp.pad(logits @ w_gate[:NC] + gate_h @ w_gate[NC:],
                 ((0, NPAD - N), (0, 0)))
    (xb0, xb1, xb2, xb3, xb4, rin_pad, rout_pad, gates, il) = _build(
        pin, pout, lg, gl, ft, dtc)

    xblocks = [xb0, xb1, xb2, xb3, xb4]
    *ublocks, usplit = _agg(NB1 - 1)(*xblocks, srcp, dstp)

    w1eff = []
    b1s = []
    w2s = []
    for e, (cfg, p) in enumerate(zip(CFGS, expert_params)):
        w1 = p["W1"]
        off = 0
        rows = jnp.zeros((C1, HID), f32)
        if "logits" in cfg:
            rows = rows.at[0:NC].set(w1[off:off + NC])
            off += NC
        if "features" in cfg:
            w1f = w1[off:off + FH]
            off += FH
            rows = rows.at[128:128 + FD].set(p["Wf"] @ w1f)
            rows = rows.at[NC].set(p["bf"] @ w1f)
        if "degrees" in cfg:
            slot = DEG_EXPERTS.index(e)
            rows = rows.at[384 + 64 * slot:384 + 64 * slot + DH].set(w1[off:off + DH])
            off += DH
        w1eff.append(rows)
        b1s.append(p["b1"])
        w2s.append(p["W2"])
    w1eff = jnp.stack(w1eff)
    b1s = jnp.stack(b1s)[None]
    w2s = jnp.stack(w2s)

    z = _stage_e(ublocks, usplit, rin_pad, rout_pad, w1eff, b1s, w2s)

    zblocks = [z[:, b * WBLK:(b + 1) * WBLK] for b in range(NB2)]
    *yblocks, ysplit = _agg(NB2 - 1)(*zblocks, srcp, dstp)

    b2s = jnp.stack([p["b2"] for p in expert_params])
    y, aux = _combine(yblocks[0], yblocks[1], ysplit, gates, rin_pad, b2s, il)
    return y, aux.reshape(())


# speedup vs baseline: 1.1182x; 1.0410x over previous
import functools

import jax
import jax.numpy as jnp
import numpy as np
from jax import lax
from jax.experimental import pallas as pl
from jax.experimental.pallas import tpu as pltpu
from jax.experimental.pallas import tpu_sc as plsc

N = 10000
NPAD = 10240
NC = 40
FD = 256
FH = 256
DH = 64
HID = 512
E = 160000
MAXD = 256
NE = 8
CFGS = [("logits", "features"), ("features",), ("logits",),
        ("logits", "features", "degrees"), ("features", "degrees"),
        ("degrees",), ("logits", "degrees"), ("logits", "features")]
DEG_EXPERTS = [3, 4, 5, 6]
FEAT_EXPERTS = [0, 1, 3, 4, 7]
C1 = 640
C2 = 384

NBLK = 1024
WBLK = 128
NB1 = C1 // WBLK
NB2 = C2 // WBLK
EPAD = 163840
NCH = EPAD // 16 // 128
TROWS = NPAD // 16


def _stage_e_body(u0, u1, u2, u3, u4s, rin_ref, rout_ref, w1_ref, b1_ref,
                  w2_ref, z_ref):
    bf16 = jnp.bfloat16
    rin = rin_ref[...]
    us = [(u0[...] * rin).astype(bf16), (u1[...] * rin).astype(bf16),
          (u2[...] * rin).astype(bf16), (u3[...] * rin).astype(bf16),
          ((u4s[0] + u4s[1]) * rin).astype(bf16)]
    outs = []
    for e in range(NE):
        h = jnp.dot(us[0], w1_ref[e, 0:WBLK].astype(bf16),
                    preferred_element_type=jnp.float32)
        for b in range(1, NB1):
            h = h + jnp.dot(us[b], w1_ref[e, b * WBLK:(b + 1) * WBLK].astype(bf16),
                            preferred_element_type=jnp.float32)
        h = jnp.maximum(h + b1_ref[0, e][None, :], 0.0)
        z = jnp.dot(h.astype(bf16), w2_ref[e].astype(bf16),
                    preferred_element_type=jnp.float32)
        outs.append(z)
    outs.append(jnp.zeros((NBLK, C2 - NE * NC), jnp.float32))
    z_ref[...] = jnp.concatenate(outs, axis=1) * rout_ref[...]


def _stage_e(ublocks, usplit, rin, rout, w1eff, b1s, w2s):
    grid = NPAD // NBLK
    ub_spec = pl.BlockSpec((NBLK, WBLK), lambda i: (i, 0))
    return pl.pallas_call(
        _stage_e_body,
        grid=(grid,),
        in_specs=[ub_spec] * (NB1 - 1) + [
            pl.BlockSpec((2, NBLK, WBLK), lambda i: (0, i, 0)),
            pl.BlockSpec((NBLK, 1), lambda i: (i, 0)),
            pl.BlockSpec((NBLK, 1), lambda i: (i, 0)),
            pl.BlockSpec((NE, C1, HID), lambda i: (0, 0, 0)),
            pl.BlockSpec((1, NE, HID), lambda i: (0, 0, 0)),
            pl.BlockSpec((NE, HID, NC), lambda i: (0, 0, 0)),
        ],
        out_specs=pl.BlockSpec((NBLK, C2), lambda i: (i, 0)),
        out_shape=jax.ShapeDtypeStruct((NPAD, C2), jnp.float32),
    )(*ublocks, usplit, rin, rout, w1eff, b1s, w2s)


def _make_agg(nfull):
    mesh = plsc.VectorSubcoreMesh(core_axis_name="c", subcore_axis_name="s")
    f32 = jnp.float32
    w = WBLK
    nblocks = nfull + 1

    @functools.partial(
        pl.kernel,
        mesh=mesh,
        out_type=[jax.ShapeDtypeStruct((NPAD, w), f32)] * nfull
        + [jax.ShapeDtypeStruct((2, NPAD, w), f32)],
        scratch_types=[
            pltpu.VMEM((NCH // 2, 64), jnp.int32),
            pltpu.VMEM((NCH // 2, 64), jnp.int32),
            pltpu.VMEM((64, w), f32),
            pltpu.VMEM((64, w), f32),
            pltpu.VMEM((64, w), f32),
            pltpu.VMEM((64, w), f32),
            pltpu.VMEM_SHARED((NPAD, w), f32),
            pltpu.SemaphoreType.DMA,
            pltpu.SemaphoreType.DMA,
        ],
    )
    def agg(*refs):
        xs = refs[:nblocks]
        srcp = refs[nblocks]
        dstp = refs[nblocks + 1]
        outs = refs[nblocks + 2:nblocks + 2 + nblocks]
        (src_v, dst_v, b0, b1, b2, b3, acc, gsem,
         ssem) = refs[nblocks + 2 + nblocks:]
        bufs = [b0, b1, b2, b3]
        cid = lax.axis_index("c")
        sid = lax.axis_index("s")
        nh = NCH // 2

        def process(x_hbm, out_rows, h_pred):
            @pl.loop(0, 64)
            def _zrow(j):
                for k in range(w // 16):
                    b0[j, pl.ds(k * 16, 16)] = jnp.zeros((16,), f32)

            for r in range(TROWS // 64):
                pltpu.sync_copy(b0,
                                acc.at[pl.ds(sid * TROWS + r * 64, 64)])
            plsc.subcore_barrier()

            for h in range(4):
                @pl.when(h_pred[h])
                def _half():
                    pltpu.sync_copy(
                        srcp.at[sid, pl.ds(h * nh, nh)], src_v)
                    pltpu.sync_copy(
                        dstp.at[sid, pl.ds(h * nh, nh)], dst_v)

                    for k in range(4):
                        pltpu.async_copy(x_hbm.at[src_v.at[k]], bufs[k],
                                         gsem)

                    @pl.loop(0, nh, step=4)
                    def _chunk(j):
                        for k in range(4):
                            pltpu.make_async_copy(
                                x_hbm.at[src_v.at[j + k]], bufs[k],
                                gsem).wait()
                            pltpu.async_copy(
                                bufs[k], acc.at[dst_v.at[j + k]], ssem,
                                add=True)
                        for k in range(4):
                            pltpu.make_async_copy(
                                bufs[k], acc.at[dst_v.at[j + k]],
                                ssem).wait()

                            @pl.when(j + k + 4 < nh)
                            def _g2():
                                pltpu.async_copy(
                                    x_hbm.at[src_v.at[j + k + 4]], bufs[k],
                                    gsem)

            plsc.subcore_barrier()
            pltpu.sync_copy(acc.at[pl.ds(sid * TROWS, TROWS)],
                            out_rows.at[pl.ds(sid * TROWS, TROWS)])
            plsc.subcore_barrier()

        for b in range(nfull):
            @pl.when(cid == b % 2)
            def _full():
                process(xs[b], outs[b], [True] * 4)

        process(xs[nfull], outs[nfull].at[cid],
                [cid == 0, cid == 0, cid == 1, cid == 1])

    return agg


_AGG_CACHE = {}


def _agg(nfull):
    if nfull not in _AGG_CACHE:
        _AGG_CACHE[nfull] = _make_agg(nfull)
    return _AGG_CACHE[nfull]


def _build_body(pin, pout, lg, gl_ref, ft, dtc, xb0, xb1, xb2, xb3,
                xb4, rin_o, rout_o, gates_o, il_o):
    f32 = jnp.float32
    i = pl.program_id(0)
    din = jnp.sum(pin[...], axis=0)
    dout = jnp.sum(pout[...], axis=0)
    rin = lax.rsqrt(jnp.maximum(din, 1.0))
    rout = lax.rsqrt(jnp.maximum(dout, 1.0))
    rin_o[...] = rin[:, None]
    rout_o[...] = rout[:, None]
    deg = jnp.clip(din + dout, 0.0, float(MAXD - 1)).astype(jnp.int32)
    oh = (deg[:, None] == lax.broadcasted_iota(jnp.int32, (NBLK, MAXD), 1)
          ).astype(f32)
    demb = jnp.dot(oh, dtc[...], preferred_element_type=f32,
                   precision=lax.Precision.HIGHEST)

    gl_ = gl_ref[...]
    iota8 = lax.broadcasted_iota(jnp.int32, (NBLK, NE), 1)
    m1 = jnp.max(gl_, axis=1, keepdims=True)
    i1 = jnp.min(jnp.where(gl_ == m1, iota8, NE), axis=1, keepdims=True)
    neg = jnp.float32(-jnp.inf)
    masked = jnp.where(iota8 == i1, neg, gl_)
    m2 = jnp.max(masked, axis=1, keepdims=True)
    i2 = jnp.min(jnp.where(masked == m2, iota8, NE), axis=1, keepdims=True)
    ex = jnp.exp(m2 - m1)
    g1 = 1.0 / (1.0 + ex)
    g2 = ex / (1.0 + ex)
    gates = jnp.where(iota8 == i1, g1, 0.0) + jnp.where(iota8 == i2, g2, 0.0)
    rowid = i * NBLK + lax.broadcasted_iota(jnp.int32, (NBLK, 1), 0)
    gates = jnp.where(rowid < N, gates, 0.0)
    gates_o[...] = gates
    pimp = jnp.sum(gates, axis=0)[None]
    pload = jnp.sum((gates > 0.0).astype(f32), axis=0)[None]
    part = jnp.concatenate([pimp, pload], axis=0)

    @pl.when(i == 0)
    def _init():
        il_o[...] = part

    @pl.when(i > 0)
    def _acc():
        il_o[...] += part

    ro = rout[:, None]
    xb0[...] = jnp.concatenate(
        [lg[...], jnp.ones((NBLK, 1), f32),
         jnp.zeros((NBLK, WBLK - NC - 1), f32)], axis=1) * ro
    xb1[...] = ft[:, 0:128] * ro
    xb2[...] = ft[:, 128:256] * ro
    xb3[...] = demb[:, 0:128] * ro
    xb4[...] = demb[:, 128:256] * ro


def _build(pin, pout, lg, gl, ft, dtc):
    f32 = jnp.float32
    grid = NPAD // NBLK
    row = lambda i: (i, 0)
    full = lambda i: (0, 0)
    return pl.pallas_call(
        _build_body,
        grid=(grid,),
        in_specs=[
            pl.BlockSpec((1, NBLK), lambda i: (0, i)),
            pl.BlockSpec((1, NBLK), lambda i: (0, i)),
            pl.BlockSpec((NBLK, NC), row),
            pl.BlockSpec((NBLK, NE), row),
            pl.BlockSpec((NBLK, FD), row),
            pl.BlockSpec((MAXD, 4 * DH), full),
        ],
        out_specs=[pl.BlockSpec((NBLK, WBLK), row)] * 5 + [
            pl.BlockSpec((NBLK, 1), row),
            pl.BlockSpec((NBLK, 1), row),
            pl.BlockSpec((NBLK, NE), row),
            pl.BlockSpec((2, NE), full),
        ],
        out_shape=[jax.ShapeDtypeStruct((NPAD, WBLK), f32)] * 5 + [
            jax.ShapeDtypeStruct((NPAD, 1), f32),
            jax.ShapeDtypeStruct((NPAD, 1), f32),
            jax.ShapeDtypeStruct((NPAD, NE), f32),
            jax.ShapeDtypeStruct((2, NE), f32),
        ],
    )(pin, pout, lg, gl, ft, dtc)


NBLK2 = 1000


def _combine_body(yb0, yb1, ys, gates, rin, b2s, il, y_o, aux_o):
    f32 = jnp.float32
    i = pl.program_id(0)
    ycat = jnp.concatenate([yb0[...], yb1[...], ys[0] + ys[1]],
                           axis=1) * rin[...]
    g = gates[...]
    acc = jnp.dot(g, b2s[...], preferred_element_type=f32)
    for e in range(NE):
        acc = acc + g[:, e:e + 1] * ycat[:, e * NC:(e + 1) * NC]
    y_o[...] = acc

    @pl.when(i == 0)
    def _aux():
        def cv2(x):
            mu = jnp.mean(x)
            var = jnp.mean(x * x) - mu * mu
            return var / (mu * mu + 1e-10)

        aux_o[...] = (0.01 * (cv2(il[0]) + cv2(il[1])))[None, None]


def _combine(yb0, yb1, ys, gates, rin, b2s, il):
    f32 = jnp.float32
    grid = N // NBLK2
    row = lambda i: (i, 0)
    full = lambda i: (0, 0)
    return pl.pallas_call(
        _combine_body,
        grid=(grid,),
        in_specs=[
            pl.BlockSpec((NBLK2, WBLK), row),
            pl.BlockSpec((NBLK2, WBLK), row),
            pl.BlockSpec((2, NBLK2, WBLK), lambda i: (0, i, 0)),
            pl.BlockSpec((NBLK2, NE), row),
            pl.BlockSpec((NBLK2, 1), row),
            pl.BlockSpec((NE, NC), full),
            pl.BlockSpec((2, NE), full),
        ],
        out_specs=[pl.BlockSpec((NBLK2, NC), row),
                   pl.BlockSpec((1, 1), full)],
        out_shape=[jax.ShapeDtypeStruct((N, NC), f32),
                   jax.ShapeDtypeStruct((1, 1), f32)],
    )(yb0, yb1, ys, gates, rin, b2s, il)


def kernel(logits, features, edge_index, gate_Wp, gate_bp, w_gate, expert_params):
    f32 = jnp.float32
    src = edge_index[0]
    dst = edge_index[1]

    srcp = jnp.pad(src, (0, EPAD - E), constant_values=N).reshape(16, 2 * NCH, 64)
    dstp = jnp.pad(dst, (0, EPAD - E), constant_values=N).reshape(16, 2 * NCH, 64)
    ones_e = jnp.ones((E,), f32)
    pin = jnp.pad(jnp.zeros((N,), f32).at[dst].add(ones_e),
                  (0, NPAD - N))[None]
    pout = jnp.pad(jnp.zeros((N,), f32).at[src].add(ones_e),
                   (0, NPAD - N))[None]

    lg = jnp.pad(logits, ((0, NPAD - N), (0, 0)))
    ft = jnp.pad(features, ((0, NPAD - N), (0, 0)))
    dtc = jnp.concatenate([expert_params[e]["deg_table"]
                           for e in DEG_EXPERTS], axis=1)
    gate_in = jnp.concatenate([logits, features @ gate_Wp + gate_bp], axis=-1)
    gl = jnp.pad(gate_in @ w_gate, ((0, NPAD - N), (0, 0)))
    (xb0, xb1, xb2, xb3, xb4, rin_pad, rout_pad, gates, il) = _build(
        pin, pout, lg, gl, ft, dtc)

    xblocks = [xb0, xb1, xb2, xb3, xb4]
    *ublocks, usplit = _agg(NB1 - 1)(*xblocks, srcp, dstp)

    w1eff = []
    b1s = []
    w2s = []
    for e, (cfg, p) in enumerate(zip(CFGS, expert_params)):
        w1 = p["W1"]
        off = 0
        rows = jnp.zeros((C1, HID), f32)
        if "logits" in cfg:
            rows = rows.at[0:NC].set(w1[off:off + NC])
            off += NC
        if "features" in cfg:
            w1f = w1[off:off + FH]
            off += FH
            rows = rows.at[128:128 + FD].set(p["Wf"] @ w1f)
            rows = rows.at[NC].set(p["bf"] @ w1f)
        if "degrees" in cfg:
            slot = DEG_EXPERTS.index(e)
            rows = rows.at[384 + 64 * slot:384 + 64 * slot + DH].set(w1[off:off + DH])
            off += DH
        w1eff.append(rows)
        b1s.append(p["b1"])
        w2s.append(p["W2"])
    w1eff = jnp.stack(w1eff)
    b1s = jnp.stack(b1s)[None]
    w2s = jnp.stack(w2s)

    z = _stage_e(ublocks, usplit, rin_pad, rout_pad, w1eff, b1s, w2s)

    zblocks = [z[:, b * WBLK:(b + 1) * WBLK] for b in range(NB2)]
    *yblocks, ysplit = _agg(NB2 - 1)(*zblocks, srcp, dstp)

    b2s = jnp.stack([p["b2"] for p in expert_params])
    y, aux = _combine(yblocks[0], yblocks[1], ysplit, gates, rin_pad, b2s, il)
    return y, aux.reshape(())
